# Initial kernel scaffold; baseline (speedup 1.0000x reference)
import jax
import jax.numpy as jnp
from jax.experimental import pallas as pl


def kernel(x, order, gh, gw):
    raise NotImplementedError("write your pallas kernel here")


# speedup vs baseline: 3.8456x; 3.8456x over previous
import jax
import jax.numpy as jnp
from jax import lax
from jax.experimental import pallas as pl
from jax.experimental.pallas import tpu as pltpu
from jax.experimental.pallas import tpu_sc as plsc

_C = 96
_NW = 32
_TASKS = 2 * (_C // 8) * 64
_PER_W = _TASKS // _NW


def _sc_body(x_hbm, order_hbm, out_hbm, order_v, i0, i1, i2, reg8,
             buf0, buf1, buf2, buf3, sem):
    cid = lax.axis_index("c")
    sid = lax.axis_index("s")
    wid = sid * 2 + cid

    pltpu.sync_copy(order_hbm, order_v)
    iota = lax.broadcasted_iota(jnp.int32, (16,), 0)

    def task_body(j, carry):
        task = wid * _PER_W + j
        b = task // 768
        r = task - b * 768
        cb = r // 64
        k = r - cb * 64

        svec = plsc.load_gather(order_v, [jnp.full((16,), b * 64 + k, jnp.int32)])
        gi = svec // 8
        gj = svec - gi * 8
        base0 = ((b * _C + cb * 8) * 384 + gi * 48) * 8 + gj
        idx_refs = (i0, i1, i2)
        for n in range(24):
            cc = n // 3
            i_blk = (n % 3) * 16
            vals = base0 + cc * 3072 + (i_blk + iota) * 8
            idx_refs[n // 8][pl.ds((n % 8) * 16, 16)] = vals
        d0 = pltpu.async_copy(x_hbm.at[i0], reg8.at[pl.ds(0, 128)], sem)
        d1 = pltpu.async_copy(x_hbm.at[i1], reg8.at[pl.ds(128, 128)], sem)
        d2 = pltpu.async_copy(x_hbm.at[i2], reg8.at[pl.ds(256, 128)], sem)
        d0.wait()
        d1.wait()
        d2.wait()

        def chunk_body(t, carry2):
            q = t // 3
            p = t - q * 3
            t2 = 143 - t
            q2 = t2 // 3
            p2 = t2 - q2 * 3
            c0 = p * 16
            mt = t // 8
            ml = (t - mt * 8) * 16
            rev = p2 * 16 + 15 - iota
            qv = jnp.full((16,), q, jnp.int32)
            q2v = jnp.full((16,), q2, jnp.int32)
            for cc in range(8):
                ro = cc * 48
                v0 = reg8[ro + q, pl.ds(c0, 16)]
                buf0[mt, cc, pl.ds(ml, 16)] = v0
                vT = plsc.load_gather(reg8, [ro + c0 + iota, qv])
                buf1[mt, cc, pl.ds(ml, 16)] = vT
                v2 = lax.rev(reg8[ro + q2, pl.ds(p2 * 16, 16)], (0,))
                buf2[mt, cc, pl.ds(ml, 16)] = v2
                v3 = plsc.load_gather(reg8, [ro + rev, q2v])
                buf3[mt, cc, pl.ds(ml, 16)] = v3
            return carry2

        lax.fori_loop(0, 144, chunk_body, 0)

        kf = 63 - k
        pltpu.sync_copy(buf0, out_hbm.at[b, 0, cb, pl.ds(k * 18, 18)])
        pltpu.sync_copy(buf1, out_hbm.at[b, 1, cb, pl.ds(k * 18, 18)])
        pltpu.sync_copy(buf2, out_hbm.at[b, 2, cb, pl.ds(kf * 18, 18)])
        pltpu.sync_copy(buf3, out_hbm.at[b, 3, cb, pl.ds(kf * 18, 18)])
        return carry

    lax.fori_loop(0, _PER_W, task_body, 0)


def kernel(x, order, gh, gw):
    B, C, H, W = x.shape
    x2 = x.reshape(B * C * H * 8, W // 8)
    ordf = order.reshape(-1)
    mesh = plsc.VectorSubcoreMesh(core_axis_name="c", subcore_axis_name="s")
    out = pl.kernel(
        _sc_body,
        out_type=jax.ShapeDtypeStruct((B, 4, C // 8, 1152, 8, 128), jnp.float32),
        mesh=mesh,
        compiler_params=pltpu.CompilerParams(
            needs_layout_passes=False, use_tc_tiling_on_sc=False
        ),
        scratch_types=[
            pltpu.VMEM((128,), jnp.int32),
            pltpu.VMEM((128,), jnp.int32),
            pltpu.VMEM((128,), jnp.int32),
            pltpu.VMEM((128,), jnp.int32),
            pltpu.VMEM((384, 48), jnp.float32),
            pltpu.VMEM((18, 8, 128), jnp.float32),
            pltpu.VMEM((18, 8, 128), jnp.float32),
            pltpu.VMEM((18, 8, 128), jnp.float32),
            pltpu.VMEM((18, 8, 128), jnp.float32),
            pltpu.SemaphoreType.DMA,
        ],
    )(x2, ordf)
    return out.transpose(0, 1, 2, 4, 3, 5).reshape(B, 4, C, H * W)


# speedup vs baseline: 6.1975x; 1.6116x over previous
import jax
import jax.numpy as jnp
from jax import lax
from jax.experimental import pallas as pl
from jax.experimental.pallas import tpu as pltpu
from jax.experimental.pallas import tpu_sc as plsc

_C = 96
_NW = 32
_TASKS = 2 * (_C // 8) * 64
_PER_W = _TASKS // _NW


def _sc_body(x_hbm, order_hbm, out_hbm, order_v,
             iA0, iA1, iA2, iB0, iB1, iB2, regA, regB,
             buf0, buf1, buf2, buf3,
             sgA, sgB, so0, so1, so2, so3):
    cid = lax.axis_index("c")
    sid = lax.axis_index("s")
    wid = sid * 2 + cid
    base = wid * _PER_W

    pltpu.sync_copy(order_hbm, order_v)
    iota = lax.broadcasted_iota(jnp.int32, (16,), 0)
    idx_a = (iA0, iA1, iA2)
    idx_b = (iB0, iB1, iB2)

    def decode(task):
        b = task // 768
        r = task - b * 768
        cb = r // 64
        k = r - cb * 64
        return b, cb, k

    def build_idx(task, ii):
        b, cb, k = decode(task)
        svec = plsc.load_gather(order_v, [jnp.full((16,), b * 64 + k, jnp.int32)])
        gi = svec // 8
        gj = svec - gi * 8
        base0 = ((b * _C + cb * 8) * 384 + gi * 48) * 8 + gj
        for n in range(24):
            cc = n // 3
            i_blk = (n % 3) * 16
            ii[n // 8][pl.ds((n % 8) * 16, 16)] = base0 + cc * 3072 + (i_blk + iota) * 8

    def gather_descs(reg, ii, sem):
        return [
            pltpu.make_async_copy(x_hbm.at[ii[q]], reg.at[pl.ds(q * 128, 128)], sem)
            for q in range(3)
        ]

    def fire_gather(reg, ii, sem):
        for d in gather_descs(reg, ii, sem):
            d.start()

    def wait_gather(reg, ii, sem):
        for d in gather_descs(reg, ii, sem):
            d.wait()

    def out_descs(task):
        b, cb, k = decode(task)
        kf = 63 - k
        return [
            pltpu.make_async_copy(buf0, out_hbm.at[b, 0, cb, pl.ds(k * 18, 18)], so0),
            pltpu.make_async_copy(buf1, out_hbm.at[b, 1, cb, pl.ds(k * 18, 18)], so1),
            pltpu.make_async_copy(buf2, out_hbm.at[b, 2, cb, pl.ds(kf * 18, 18)], so2),
            pltpu.make_async_copy(buf3, out_hbm.at[b, 3, cb, pl.ds(kf * 18, 18)], so3),
        ]

    def compute_task(task, reg, first=False):
        descs = out_descs(task)
        if not first:
            for d in descs:
                d.wait()

        def chunk_body(t, carry):
            q = t // 3
            p = t - q * 3
            c0 = p * 16
            mt = t // 8
            ml = (t - mt * 8) * 16
            tr = 143 - t
            mtr = tr // 8
            mlr = (tr - mtr * 8) * 16
            qv = jnp.full((16,), q, jnp.int32)
            for cc in range(8):
                ro = cc * 48
                v0 = reg[ro + q, pl.ds(c0, 16)]
                buf0[mt, cc, pl.ds(ml, 16)] = v0
                buf2[mtr, cc, pl.ds(mlr, 16)] = lax.rev(v0, (0,))
                vT = plsc.load_gather(reg, [ro + c0 + iota, qv])
                buf1[mt, cc, pl.ds(ml, 16)] = vT
                buf3[mtr, cc, pl.ds(mlr, 16)] = lax.rev(vT, (0,))
            return carry

        lax.fori_loop(0, 144, chunk_body, 0, unroll=2)
        for d in descs:
            d.start()

    build_idx(base, idx_a)
    fire_gather(regA, idx_a, sgA)
    build_idx(base + 1, idx_b)
    fire_gather(regB, idx_b, sgB)
    wait_gather(regA, idx_a, sgA)
    compute_task(base, regA, first=True)
    build_idx(base + 2, idx_a)
    fire_gather(regA, idx_a, sgA)
    wait_gather(regB, idx_b, sgB)
    compute_task(base + 1, regB)

    def pipe_body(i, carry):
        tA = base + 2 * i
        tB = tA + 1
        build_idx(tB, idx_b)
        fire_gather(regB, idx_b, sgB)
        wait_gather(regA, idx_a, sgA)
        compute_task(tA, regA)
        nxt = base + jnp.minimum(2 * i + 2, _PER_W - 1)
        build_idx(nxt, idx_a)
        fire_gather(regA, idx_a, sgA)
        wait_gather(regB, idx_b, sgB)
        compute_task(tB, regB)
        return carry

    lax.fori_loop(1, _PER_W // 2, pipe_body, 0)

    wait_gather(regA, idx_a, sgA)
    for d in out_descs(base + _PER_W - 1):
        d.wait()


def kernel(x, order, gh, gw):
    B, C, H, W = x.shape
    x2 = x.reshape(B * C * H * 8, W // 8)
    ordf = order.reshape(-1)
    mesh = plsc.VectorSubcoreMesh(core_axis_name="c", subcore_axis_name="s")
    out = pl.kernel(
        _sc_body,
        out_type=jax.ShapeDtypeStruct((B, 4, C // 8, 1152, 8, 128), jnp.float32),
        mesh=mesh,
        compiler_params=pltpu.CompilerParams(
            needs_layout_passes=False, use_tc_tiling_on_sc=False
        ),
        scratch_types=[
            pltpu.VMEM((128,), jnp.int32),
            pltpu.VMEM((128,), jnp.int32),
            pltpu.VMEM((128,), jnp.int32),
            pltpu.VMEM((128,), jnp.int32),
            pltpu.VMEM((128,), jnp.int32),
            pltpu.VMEM((128,), jnp.int32),
            pltpu.VMEM((128,), jnp.int32),
            pltpu.VMEM((384, 48), jnp.float32),
            pltpu.VMEM((384, 48), jnp.float32),
            pltpu.VMEM((18, 8, 128), jnp.float32),
            pltpu.VMEM((18, 8, 128), jnp.float32),
            pltpu.VMEM((18, 8, 128), jnp.float32),
            pltpu.VMEM((18, 8, 128), jnp.float32),
            pltpu.SemaphoreType.DMA,
            pltpu.SemaphoreType.DMA,
            pltpu.SemaphoreType.DMA,
            pltpu.SemaphoreType.DMA,
            pltpu.SemaphoreType.DMA,
            pltpu.SemaphoreType.DMA,
        ],
    )(x2, ordf)
    return out.transpose(0, 1, 2, 4, 3, 5).reshape(B, 4, C, H * W)
